# baseline (speedup 1.0000x reference)
import functools

import jax
import jax.numpy as jnp
from jax import lax
from jax.experimental import pallas as pl
from jax.experimental.pallas import tpu as pltpu
from jax.experimental.pallas import tpu_sc as plsc

NC = 1
NS = 16
NW = NC * NS
CHUNK = 2048
CROWS = CHUNK // 128


def _worker_id():
    c = lax.axis_index("c")
    s = lax.axis_index("s")
    return s * NC + c, c, s


def _make_deg_kernel(NP, EP):
    epw = EP // NW
    nch = epw // CHUNK
    mesh = plsc.VectorSubcoreMesh(
        core_axis_name="c", subcore_axis_name="s", num_cores=NC
    )

    @functools.partial(
        pl.kernel,
        mesh=mesh,
        out_type=jax.ShapeDtypeStruct((NC * NP,), jnp.float32),
        scratch_types=[
            pltpu.VMEM((CROWS, 128), jnp.int32),
            pltpu.VMEM((CHUNK,), jnp.float32),
            pltpu.VMEM_SHARED((NP,), jnp.float32),
            pltpu.SemaphoreType.DMA,
        ],
    )
    def deg_kernel(row_hbm, w_hbm, z1_hbm, out_hbm, rowb, wb, accum, sem):
        wid, c, s = _worker_id()

        @pl.when(s == 0)
        def _():
            pltpu.sync_copy(z1_hbm, accum)

        plsc.subcore_barrier()
        for k in range(nch):
            rbase = wid * (epw // 128) + k * CROWS
            ebase = wid * epw + k * CHUNK
            pltpu.sync_copy(row_hbm.at[pl.ds(rbase, CROWS)], rowb)
            pltpu.sync_copy(w_hbm.at[pl.ds(ebase, CHUNK)], wb)
            cps = [
                pltpu.async_copy(
                    wb.at[pl.ds(j * 128, 128)], accum.at[rowb.at[j]], sem, add=True
                )
                for j in range(CROWS)
            ]
            for cp in cps:
                cp.wait()
        plsc.subcore_barrier()

        @pl.when(s == 0)
        def _():
            pltpu.sync_copy(accum, out_hbm.at[pl.ds(c * NP, NP)])

    return deg_kernel


def _make_edge_kernel(NP, EP):
    epw = EP // NW
    nch = epw // CHUNK
    rps = NP // NS
    mesh = plsc.VectorSubcoreMesh(
        core_axis_name="c", subcore_axis_name="s", num_cores=NC
    )

    @functools.partial(
        pl.kernel,
        mesh=mesh,
        out_type=jax.ShapeDtypeStruct((NC * NP, 8), jnp.float32),
        scratch_types=[
            pltpu.VMEM((CROWS, 128), jnp.int32),
            pltpu.VMEM((CROWS, 128), jnp.int32),
            pltpu.VMEM((CHUNK,), jnp.float32),
            pltpu.VMEM((CHUNK, 8), jnp.float32),
            pltpu.VMEM_SHARED((NP, 8), jnp.float32),
            pltpu.VMEM_SHARED((NP, 8), jnp.float32),
            pltpu.SemaphoreType.DMA,
        ],
        compiler_params=pltpu.CompilerParams(
            needs_layout_passes=False, use_tc_tiling_on_sc=False
        ),
    )
    def edge_kernel(
        y_hbm, col_hbm, row_hbm, w_hbm, z8_hbm, out_hbm,
        colb, rowb, wb, msgs, table, accum, sem,
    ):
        wid, c, s = _worker_id()
        rs = s * rps
        pltpu.sync_copy(y_hbm.at[pl.ds(rs, rps)], table.at[pl.ds(rs, rps)])
        pltpu.sync_copy(z8_hbm.at[pl.ds(rs, rps)], accum.at[pl.ds(rs, rps)])
        plsc.subcore_barrier()

        lanes = lax.iota(jnp.int32, 16)
        pat01 = jnp.where(lanes >= 8, 1, 0)
        fvec = lanes & 7

        for k in range(nch):
            rbase = wid * (epw // 128) + k * CROWS
            ebase = wid * epw + k * CHUNK
            pltpu.sync_copy(col_hbm.at[pl.ds(rbase, CROWS)], colb)
            pltpu.sync_copy(row_hbm.at[pl.ds(rbase, CROWS)], rowb)
            pltpu.sync_copy(w_hbm.at[pl.ds(ebase, CHUNK)], wb)
            cps = [
                pltpu.async_copy(
                    table.at[colb.at[j]], msgs.at[pl.ds(j * 128, 128)], sem
                )
                for j in range(CROWS)
            ]
            for cp in cps:
                cp.wait()

            @pl.loop(0, CHUNK // 2, unroll=8)
            def _(v):
                e = pat01 + 2 * v
                wv = plsc.load_gather(wb, [e])
                m = plsc.load_gather(msgs, [e, fvec])
                plsc.store_scatter(msgs, [e, fvec], m * wv)

            cps = [
                pltpu.async_copy(
                    msgs.at[pl.ds(j * 128, 128)], accum.at[rowb.at[j]], sem, add=True
                )
                for j in range(CROWS)
            ]
            for cp in cps:
                cp.wait()
        plsc.subcore_barrier()
        pltpu.sync_copy(accum.at[pl.ds(rs, rps)], out_hbm.at[pl.ds(c * NP + rs, rps)])

    return edge_kernel


def _tc_pre(x, w1p, d0, d1):
    N = x.shape[0]

    def body(x_ref, w1_ref, d0_ref, d1_ref, y_ref, dinv_ref):
        deg = d0_ref[...] + d1_ref[...] + 1.0
        dinv = jnp.where(deg > 0, lax.rsqrt(deg), 0.0)
        xw = jnp.dot(x_ref[...], w1_ref[...], preferred_element_type=jnp.float32)
        y_ref[...] = xw * dinv
        dinv_ref[...] = dinv

    return pl.pallas_call(
        body,
        out_shape=[
            jax.ShapeDtypeStruct((N, 8), jnp.float32),
            jax.ShapeDtypeStruct((N, 1), jnp.float32),
        ],
    )(x, w1p, d0, d1)


def _tc_mid(s0, s1, y, dinv, b1p, w2p):
    N = y.shape[0]

    def body(s0_ref, s1_ref, y_ref, dinv_ref, b_ref, w2_ref, y2_ref):
        dinv = dinv_ref[...]
        h = (s0_ref[...] + s1_ref[...] + y_ref[...]) * dinv + b_ref[...]
        y2_ref[...] = (
            jnp.dot(h, w2_ref[...], preferred_element_type=jnp.float32) * dinv
        )

    return pl.pallas_call(
        body, out_shape=jax.ShapeDtypeStruct((N, 8), jnp.float32)
    )(s0, s1, y, dinv, b1p, w2p)


def _tc_post(s0, s1, y, dinv, b2p):
    N = y.shape[0]

    def body(s0_ref, s1_ref, y_ref, dinv_ref, b_ref, o_ref):
        o_ref[...] = (
            s0_ref[...] + s1_ref[...] + y_ref[...]
        ) * dinv_ref[...] + b_ref[...]

    return pl.pallas_call(
        body, out_shape=jax.ShapeDtypeStruct((N, 8), jnp.float32)
    )(s0, s1, y, dinv, b2p)


def kernel(x, edge_index, edge_attr, adj, W1, b1, W2, b2):
    N, D = x.shape
    E = edge_index.shape[1]
    f32 = jnp.float32

    EP = -(-E // (NW * CHUNK)) * (NW * CHUNK)
    NP = -(-N // (NS * 128)) * (NS * 128)
    pad = EP - E
    ei = edge_index.astype(jnp.int32)
    row = jnp.concatenate([ei[0], jnp.zeros((pad,), jnp.int32)]).reshape(-1, 128)
    col = jnp.concatenate([ei[1], jnp.zeros((pad,), jnp.int32)]).reshape(-1, 128)
    w = jnp.concatenate([edge_attr.reshape(-1).astype(f32), jnp.zeros((pad,), f32)])

    w1p = jnp.pad(W1.astype(f32), ((0, 0), (0, 8 - W1.shape[1])))
    w2p = jnp.pad(W2.astype(f32), ((0, 8 - W2.shape[0]), (0, 8 - W2.shape[1])))
    b1p = jnp.pad(b1.astype(f32), (0, 8 - b1.shape[0])).reshape(1, 8)
    b2p = jnp.pad(b2.astype(f32), (0, 8 - b2.shape[0])).reshape(1, 8)
    z1 = jnp.zeros((NP,), f32)
    z8 = jnp.zeros((NP, 8), f32)

    degp = _make_deg_kernel(NP, EP)(row, w, z1)
    d0 = degp[:N].reshape(N, 1)
    d1 = degp[NP:NP + N].reshape(N, 1) if NC == 2 else jnp.zeros((N, 1), f32)

    y1, dinv = _tc_pre(x.astype(f32), w1p, d0, d1)
    edge_k = _make_edge_kernel(NP, EP)
    npad = ((0, NP - N), (0, 0))
    zn8 = jnp.zeros((N, 8), f32)
    S1 = edge_k(jnp.pad(y1, npad), col, row, w, z8)
    S1b = S1[NP:NP + N] if NC == 2 else zn8
    y2 = _tc_mid(S1[:N], S1b, y1, dinv, b1p, w2p)

    S2 = edge_k(jnp.pad(y2, npad), col, row, w, z8)
    S2b = S2[NP:NP + N] if NC == 2 else zn8
    out8 = _tc_post(S2[:N], S2b, y2, dinv, b2p)

    h = out8[:, :7]
    reg = jnp.array([0.0], dtype=h.dtype)
    return (h, reg)


# speedup vs baseline: 1.4251x; 1.4251x over previous
import functools

import jax
import jax.numpy as jnp
from jax import lax
from jax.experimental import pallas as pl
from jax.experimental.pallas import tpu as pltpu
from jax.experimental.pallas import tpu_sc as plsc

NC = 2
NS = 16
NW = NC * NS
CHUNK = 2048
CROWS = CHUNK // 128


def _worker_id():
    c = lax.axis_index("c")
    s = lax.axis_index("s")
    return s * NC + c, c, s


def _make_deg_kernel(NP, EP):
    epw = EP // NW
    nch = epw // CHUNK
    mesh = plsc.VectorSubcoreMesh(
        core_axis_name="c", subcore_axis_name="s", num_cores=NC
    )

    @functools.partial(
        pl.kernel,
        mesh=mesh,
        out_type=jax.ShapeDtypeStruct((NC * NP,), jnp.float32),
        scratch_types=[
            pltpu.VMEM((CROWS, 128), jnp.int32),
            pltpu.VMEM((CHUNK,), jnp.float32),
            pltpu.VMEM_SHARED((NP,), jnp.float32),
            pltpu.SemaphoreType.DMA,
        ],
    )
    def deg_kernel(row_hbm, w_hbm, z1_hbm, out_hbm, rowb, wb, accum, sem):
        wid, c, s = _worker_id()

        @pl.when(s == 0)
        def _():
            pltpu.sync_copy(z1_hbm, accum)

        plsc.subcore_barrier()
        for k in range(nch):
            rbase = wid * (epw // 128) + k * CROWS
            ebase = wid * epw + k * CHUNK
            pltpu.sync_copy(row_hbm.at[pl.ds(rbase, CROWS)], rowb)
            pltpu.sync_copy(w_hbm.at[pl.ds(ebase, CHUNK)], wb)
            cps = [
                pltpu.async_copy(
                    wb.at[pl.ds(j * 128, 128)], accum.at[rowb.at[j]], sem, add=True
                )
                for j in range(CROWS)
            ]
            for cp in cps:
                cp.wait()
        plsc.subcore_barrier()

        @pl.when(s == 0)
        def _():
            pltpu.sync_copy(accum, out_hbm.at[pl.ds(c * NP, NP)])

    return deg_kernel


def _make_edge_kernel(NP, EP):
    epw = EP // NW
    nch = epw // CHUNK
    rps = NP // NS
    mesh = plsc.VectorSubcoreMesh(
        core_axis_name="c", subcore_axis_name="s", num_cores=NC
    )

    @functools.partial(
        pl.kernel,
        mesh=mesh,
        out_type=jax.ShapeDtypeStruct((NC * NP, 8), jnp.float32),
        scratch_types=[
            [pltpu.VMEM((CROWS, 128), jnp.int32) for _ in range(3)],
            [pltpu.VMEM((CROWS, 128), jnp.int32) for _ in range(3)],
            [pltpu.VMEM((CHUNK,), jnp.float32) for _ in range(3)],
            [pltpu.VMEM((CHUNK, 8), jnp.float32) for _ in range(2)],
            pltpu.VMEM_SHARED((NP, 8), jnp.float32),
            pltpu.VMEM_SHARED((NP, 8), jnp.float32),
            [pltpu.SemaphoreType.DMA for _ in range(3)],
            [pltpu.SemaphoreType.DMA for _ in range(2)],
            [pltpu.SemaphoreType.DMA for _ in range(2)],
            pltpu.SemaphoreType.DMA,
        ],
        compiler_params=pltpu.CompilerParams(
            needs_layout_passes=False, use_tc_tiling_on_sc=False
        ),
    )
    def edge_kernel(
        y_hbm, col_hbm, row_hbm, w_hbm, z8_hbm, out_hbm,
        colb, rowb, wb, msgs, table, accum, si, sg, ss, st,
    ):
        wid, c, s = _worker_id()
        rs = s * rps
        a_tab = pltpu.async_copy(
            y_hbm.at[pl.ds(rs, rps)], table.at[pl.ds(rs, rps)], st
        )
        a_z = pltpu.async_copy(
            z8_hbm.at[pl.ds(rs, rps)], accum.at[pl.ds(rs, rps)], st
        )

        def stage(k):
            i3 = k % 3
            rbase = wid * (epw // 128) + k * CROWS
            ebase = wid * epw + k * CHUNK
            return [
                pltpu.async_copy(col_hbm.at[pl.ds(rbase, CROWS)], colb[i3], si[i3]),
                pltpu.async_copy(row_hbm.at[pl.ds(rbase, CROWS)], rowb[i3], si[i3]),
                pltpu.async_copy(w_hbm.at[pl.ds(ebase, CHUNK)], wb[i3], si[i3]),
            ]

        def fire_gathers(k):
            i3, i2 = k % 3, k % 2
            return [
                pltpu.async_copy(
                    table.at[colb[i3].at[j]],
                    msgs[i2].at[pl.ds(j * 128, 128)],
                    sg[i2],
                )
                for j in range(CROWS)
            ]

        def fire_scatters(k):
            i3, i2 = k % 3, k % 2
            return [
                pltpu.async_copy(
                    msgs[i2].at[pl.ds(j * 128, 128)],
                    accum.at[rowb[i3].at[j]],
                    ss[i2],
                    add=True,
                )
                for j in range(CROWS)
            ]

        def drain(cps):
            for cp in cps:
                cp.wait()

        lanes = lax.iota(jnp.int32, 16)
        pat01 = jnp.where(lanes >= 8, 1, 0)
        fvec = lanes & 7

        def mult(k):
            i3, i2 = k % 3, k % 2

            @pl.loop(0, CHUNK // 2, unroll=8)
            def _(v):
                e = pat01 + 2 * v
                wv = plsc.load_gather(wb[i3], [e])
                m = plsc.load_gather(msgs[i2], [e, fvec])
                plsc.store_scatter(msgs[i2], [e, fvec], m * wv)

        i_cps = [None] * nch
        g_cps = [None] * nch
        s_cps = [None] * nch
        i_cps[0] = stage(0)
        if nch > 1:
            i_cps[1] = stage(1)
        a_tab.wait()
        a_z.wait()
        plsc.subcore_barrier()
        drain(i_cps[0])
        g_cps[0] = fire_gathers(0)

        for k in range(nch):
            drain(g_cps[k])
            mult(k)
            s_cps[k] = fire_scatters(k)
            if k + 1 < nch:
                drain(i_cps[k + 1])
                if k >= 1:
                    drain(s_cps[k - 1])
                g_cps[k + 1] = fire_gathers(k + 1)
                if k + 2 < nch:
                    i_cps[k + 2] = stage(k + 2)
        if nch >= 2:
            drain(s_cps[nch - 2])
        drain(s_cps[nch - 1])
        plsc.subcore_barrier()
        pltpu.sync_copy(accum.at[pl.ds(rs, rps)], out_hbm.at[pl.ds(c * NP + rs, rps)])

    return edge_kernel


def _tc_pre(x, w1p, d0, d1):
    N = x.shape[0]

    def body(x_ref, w1_ref, d0_ref, d1_ref, y_ref, dinv_ref):
        deg = d0_ref[...] + d1_ref[...] + 1.0
        dinv = jnp.where(deg > 0, lax.rsqrt(deg), 0.0)
        xw = jnp.dot(x_ref[...], w1_ref[...], preferred_element_type=jnp.float32)
        y_ref[...] = xw * dinv
        dinv_ref[...] = dinv

    return pl.pallas_call(
        body,
        out_shape=[
            jax.ShapeDtypeStruct((N, 8), jnp.float32),
            jax.ShapeDtypeStruct((N, 1), jnp.float32),
        ],
    )(x, w1p, d0, d1)


def _tc_mid(s0, s1, y, dinv, b1p, w2p):
    N = y.shape[0]

    def body(s0_ref, s1_ref, y_ref, dinv_ref, b_ref, w2_ref, y2_ref):
        dinv = dinv_ref[...]
        h = (s0_ref[...] + s1_ref[...] + y_ref[...]) * dinv + b_ref[...]
        y2_ref[...] = (
            jnp.dot(h, w2_ref[...], preferred_element_type=jnp.float32) * dinv
        )

    return pl.pallas_call(
        body, out_shape=jax.ShapeDtypeStruct((N, 8), jnp.float32)
    )(s0, s1, y, dinv, b1p, w2p)


def _tc_post(s0, s1, y, dinv, b2p):
    N = y.shape[0]

    def body(s0_ref, s1_ref, y_ref, dinv_ref, b_ref, o_ref):
        o_ref[...] = (
            s0_ref[...] + s1_ref[...] + y_ref[...]
        ) * dinv_ref[...] + b_ref[...]

    return pl.pallas_call(
        body, out_shape=jax.ShapeDtypeStruct((N, 8), jnp.float32)
    )(s0, s1, y, dinv, b2p)


def kernel(x, edge_index, edge_attr, adj, W1, b1, W2, b2):
    N, D = x.shape
    E = edge_index.shape[1]
    f32 = jnp.float32

    EP = -(-E // (NW * CHUNK)) * (NW * CHUNK)
    NP = -(-N // (NS * 128)) * (NS * 128)
    pad = EP - E
    ei = edge_index.astype(jnp.int32)
    row = jnp.concatenate([ei[0], jnp.zeros((pad,), jnp.int32)]).reshape(-1, 128)
    col = jnp.concatenate([ei[1], jnp.zeros((pad,), jnp.int32)]).reshape(-1, 128)
    w = jnp.concatenate([edge_attr.reshape(-1).astype(f32), jnp.zeros((pad,), f32)])

    w1p = jnp.pad(W1.astype(f32), ((0, 0), (0, 8 - W1.shape[1])))
    w2p = jnp.pad(W2.astype(f32), ((0, 8 - W2.shape[0]), (0, 8 - W2.shape[1])))
    b1p = jnp.pad(b1.astype(f32), (0, 8 - b1.shape[0])).reshape(1, 8)
    b2p = jnp.pad(b2.astype(f32), (0, 8 - b2.shape[0])).reshape(1, 8)
    z1 = jnp.zeros((NP,), f32)
    z8 = jnp.zeros((NP, 8), f32)

    degp = _make_deg_kernel(NP, EP)(row, w, z1)
    d0 = degp[:N].reshape(N, 1)
    d1 = degp[NP:NP + N].reshape(N, 1) if NC == 2 else jnp.zeros((N, 1), f32)

    y1, dinv = _tc_pre(x.astype(f32), w1p, d0, d1)
    edge_k = _make_edge_kernel(NP, EP)
    npad = ((0, NP - N), (0, 0))
    zn8 = jnp.zeros((N, 8), f32)
    S1 = edge_k(jnp.pad(y1, npad), col, row, w, z8)
    S1b = S1[NP:NP + N] if NC == 2 else zn8
    y2 = _tc_mid(S1[:N], S1b, y1, dinv, b1p, w2p)

    S2 = edge_k(jnp.pad(y2, npad), col, row, w, z8)
    S2b = S2[NP:NP + N] if NC == 2 else zn8
    out8 = _tc_post(S2[:N], S2b, y2, dinv, b2p)

    h = out8[:, :7]
    reg = jnp.array([0.0], dtype=h.dtype)
    return (h, reg)


# speedup vs baseline: 1.9449x; 1.3648x over previous
import functools

import jax
import jax.numpy as jnp
from jax import lax
from jax.experimental import pallas as pl
from jax.experimental.pallas import tpu as pltpu
from jax.experimental.pallas import tpu_sc as plsc

NC = 2
NS = 16
NW = NC * NS
CHUNK = 2048


def _worker_id():
    c = lax.axis_index("c")
    s = lax.axis_index("s")
    return s * NC + c, c, s


def _make_deg_kernel(NP, EP):
    epw = EP // NW
    nch = epw // CHUNK
    mesh = plsc.VectorSubcoreMesh(
        core_axis_name="c", subcore_axis_name="s", num_cores=NC
    )

    @functools.partial(
        pl.kernel,
        mesh=mesh,
        out_type=jax.ShapeDtypeStruct((NC * NP,), jnp.float32),
        scratch_types=[
            [pltpu.VMEM((CHUNK,), jnp.int32) for _ in range(3)],
            [pltpu.VMEM((CHUNK,), jnp.float32) for _ in range(3)],
            pltpu.VMEM_SHARED((NP,), jnp.float32),
            [pltpu.SemaphoreType.DMA for _ in range(3)],
            [pltpu.SemaphoreType.DMA for _ in range(2)],
            pltpu.SemaphoreType.DMA,
        ],
    )
    def deg_kernel(row_hbm, w_hbm, z1_hbm, out_hbm, rowb, wb, accum, si, ss, st):
        wid, c, s = _worker_id()

        def stage(k):
            i3 = k % 3
            ebase = wid * epw + k * CHUNK
            return [
                pltpu.async_copy(row_hbm.at[pl.ds(ebase, CHUNK)], rowb[i3], si[i3]),
                pltpu.async_copy(w_hbm.at[pl.ds(ebase, CHUNK)], wb[i3], si[i3]),
            ]

        def drain(cps):
            for cp in cps:
                cp.wait()

        @pl.when(s == 0)
        def _():
            pltpu.async_copy(z1_hbm, accum, st)

        i_cps = [None] * nch
        s_cps = [None] * nch
        i_cps[0] = stage(0)
        if nch > 1:
            i_cps[1] = stage(1)

        @pl.when(s == 0)
        def _():
            pltpu.make_async_copy(z1_hbm, accum, st).wait()

        plsc.subcore_barrier()
        for k in range(nch):
            i3 = k % 3
            drain(i_cps[k])
            if k >= 1:
                drain(s_cps[k - 1])
            s_cps[k] = [
                pltpu.async_copy(wb[i3], accum.at[rowb[i3]], ss[k % 2], add=True)
            ]
            if k + 2 < nch:
                i_cps[k + 2] = stage(k + 2)
        drain(s_cps[nch - 1])
        plsc.subcore_barrier()

        @pl.when(s == 0)
        def _():
            pltpu.sync_copy(accum, out_hbm.at[pl.ds(c * NP, NP)])

    return deg_kernel


def _make_edge_kernel(NP, EP):
    epw = EP // NW
    nch = epw // CHUNK
    rps = NP // NS
    mesh = plsc.VectorSubcoreMesh(
        core_axis_name="c", subcore_axis_name="s", num_cores=NC
    )

    @functools.partial(
        pl.kernel,
        mesh=mesh,
        out_type=jax.ShapeDtypeStruct((NC * NP, 8), jnp.float32),
        scratch_types=[
            [pltpu.VMEM((CHUNK,), jnp.int32) for _ in range(3)],
            [pltpu.VMEM((CHUNK,), jnp.int32) for _ in range(3)],
            [pltpu.VMEM((CHUNK,), jnp.float32) for _ in range(3)],
            [pltpu.VMEM((CHUNK, 8), jnp.float32) for _ in range(2)],
            pltpu.VMEM_SHARED((NP, 8), jnp.float32),
            pltpu.VMEM_SHARED((NP, 8), jnp.float32),
            [pltpu.SemaphoreType.DMA for _ in range(3)],
            [pltpu.SemaphoreType.DMA for _ in range(2)],
            [pltpu.SemaphoreType.DMA for _ in range(2)],
            pltpu.SemaphoreType.DMA,
        ],
        compiler_params=pltpu.CompilerParams(
            needs_layout_passes=False, use_tc_tiling_on_sc=False
        ),
    )
    def edge_kernel(
        y_hbm, col_hbm, row_hbm, w_hbm, z8_hbm, out_hbm,
        colb, rowb, wb, msgs, table, accum, si, sg, ss, st,
    ):
        wid, c, s = _worker_id()
        rs = s * rps
        a_tab = pltpu.async_copy(
            y_hbm.at[pl.ds(rs, rps)], table.at[pl.ds(rs, rps)], st
        )
        a_z = pltpu.async_copy(
            z8_hbm.at[pl.ds(rs, rps)], accum.at[pl.ds(rs, rps)], st
        )

        def stage(k):
            i3 = k % 3
            ebase = wid * epw + k * CHUNK
            return [
                pltpu.async_copy(col_hbm.at[pl.ds(ebase, CHUNK)], colb[i3], si[i3]),
                pltpu.async_copy(row_hbm.at[pl.ds(ebase, CHUNK)], rowb[i3], si[i3]),
                pltpu.async_copy(w_hbm.at[pl.ds(ebase, CHUNK)], wb[i3], si[i3]),
            ]

        def fire_gathers(k):
            i3, i2 = k % 3, k % 2
            return [pltpu.async_copy(table.at[colb[i3]], msgs[i2], sg[i2])]

        def fire_scatters(k):
            i3, i2 = k % 3, k % 2
            return [
                pltpu.async_copy(msgs[i2], accum.at[rowb[i3]], ss[i2], add=True)
            ]

        def drain(cps):
            for cp in cps:
                cp.wait()

        lanes = lax.iota(jnp.int32, 16)
        pat01 = jnp.where(lanes >= 8, 1, 0)
        fvec = lanes & 7

        def mult(k):
            i3, i2 = k % 3, k % 2

            @pl.loop(0, CHUNK // 2, unroll=8)
            def _(v):
                e = pat01 + 2 * v
                wv = plsc.load_gather(wb[i3], [e])
                m = plsc.load_gather(msgs[i2], [e, fvec])
                plsc.store_scatter(msgs[i2], [e, fvec], m * wv)

        i_cps = [None] * nch
        g_cps = [None] * nch
        s_cps = [None] * nch
        i_cps[0] = stage(0)
        if nch > 1:
            i_cps[1] = stage(1)
        a_tab.wait()
        a_z.wait()
        plsc.subcore_barrier()
        drain(i_cps[0])
        g_cps[0] = fire_gathers(0)

        for k in range(nch):
            drain(g_cps[k])
            mult(k)
            s_cps[k] = fire_scatters(k)
            if k + 1 < nch:
                drain(i_cps[k + 1])
                if k >= 1:
                    drain(s_cps[k - 1])
                g_cps[k + 1] = fire_gathers(k + 1)
                if k + 2 < nch:
                    i_cps[k + 2] = stage(k + 2)
        if nch >= 2:
            drain(s_cps[nch - 2])
        drain(s_cps[nch - 1])
        plsc.subcore_barrier()
        pltpu.sync_copy(accum.at[pl.ds(rs, rps)], out_hbm.at[pl.ds(c * NP + rs, rps)])

    return edge_kernel


def _tc_prep(ei, ea_t, EP, E):
    G = 5
    BLK = EP // G

    def body(ei_ref, ea_ref, row_ref, col_ref, w_ref):
        base = pl.program_id(0) * BLK
        pos = lax.iota(jnp.int32, BLK)
        valid = (base + pos) < E
        row_ref[...] = jnp.where(valid, ei_ref[0, :], 0)
        col_ref[...] = jnp.where(valid, ei_ref[1, :], 0)
        w_ref[...] = jnp.where(valid, ea_ref[0, :], 0.0)

    return pl.pallas_call(
        body,
        grid=(G,),
        in_specs=[
            pl.BlockSpec((2, BLK), lambda i: (0, i)),
            pl.BlockSpec((1, BLK), lambda i: (0, i)),
        ],
        out_specs=[
            pl.BlockSpec((BLK,), lambda i: (i,)),
            pl.BlockSpec((BLK,), lambda i: (i,)),
            pl.BlockSpec((BLK,), lambda i: (i,)),
        ],
        out_shape=[
            jax.ShapeDtypeStruct((EP,), jnp.int32),
            jax.ShapeDtypeStruct((EP,), jnp.int32),
            jax.ShapeDtypeStruct((EP,), jnp.float32),
        ],
    )(ei, ea_t)


def _dinv_t(degp_ref, N, NP):
    dp = degp_ref[...]
    deg = dp[0:NP] + dp[NP:2 * NP] + 1.0
    dinv = jnp.where(deg > 0, lax.rsqrt(deg), 0.0)
    return dinv.reshape(1, NP)


def _tc_pre(x, w1p, degp, NP):
    N = x.shape[0]

    def body(x_ref, w1_ref, degp_ref, y_ref):
        dinv = _dinv_t(degp_ref, N, NP)
        xwt = lax.dot_general(
            w1_ref[...], x_ref[...], (((0,), (1,)), ((), ())),
            preferred_element_type=jnp.float32,
        )
        y_ref[...] = jnp.pad(xwt, ((0, 0), (0, NP - N))) * dinv

    return pl.pallas_call(
        body, out_shape=jax.ShapeDtypeStruct((8, NP), jnp.float32)
    )(x, w1p, degp)


def _tc_mid(st, yt, degp, b1p, w2p, NP):
    N = NP

    def body(st_ref, yt_ref, degp_ref, b_ref, w2_ref, y2_ref):
        dinv = _dinv_t(degp_ref, N, NP)
        stv = st_ref[...]
        h = (stv[:, 0:NP] + stv[:, NP:2 * NP] + yt_ref[...]) * dinv
        h = h + b_ref[...]
        y2_ref[...] = (
            lax.dot_general(
                w2_ref[...], h, (((0,), (0,)), ((), ())),
                preferred_element_type=jnp.float32,
            )
            * dinv
        )

    return pl.pallas_call(
        body, out_shape=jax.ShapeDtypeStruct((8, NP), jnp.float32)
    )(st, yt, degp, b1p, w2p)


def _tc_post(st, yt, degp, b2p, NP):
    N = NP

    def body(st_ref, yt_ref, degp_ref, b_ref, o_ref):
        dinv = _dinv_t(degp_ref, N, NP)
        stv = st_ref[...]
        o_ref[...] = (
            stv[:, 0:NP] + stv[:, NP:2 * NP] + yt_ref[...]
        ) * dinv + b_ref[...]

    return pl.pallas_call(
        body, out_shape=jax.ShapeDtypeStruct((8, NP), jnp.float32)
    )(st, yt, degp, b2p)


def kernel(x, edge_index, edge_attr, adj, W1, b1, W2, b2):
    N, D = x.shape
    E = edge_index.shape[1]
    f32 = jnp.float32

    EP = -(-E // (NW * CHUNK)) * (NW * CHUNK)
    NP = -(-N // (NS * 128)) * (NS * 128)
    ei = edge_index.astype(jnp.int32)
    ea_t = edge_attr.astype(f32).reshape(1, E)
    row, col, w = _tc_prep(ei, ea_t, EP, E)

    w1p = jnp.pad(W1.astype(f32), ((0, 0), (0, 8 - W1.shape[1])))
    w2p = jnp.pad(W2.astype(f32), ((0, 8 - W2.shape[0]), (0, 8 - W2.shape[1])))
    b1p = jnp.pad(b1.astype(f32), (0, 8 - b1.shape[0])).reshape(8, 1)
    b2p = jnp.pad(b2.astype(f32), (0, 8 - b2.shape[0])).reshape(8, 1)
    z1 = jnp.zeros((NP,), f32)
    z8 = jnp.zeros((NP, 8), f32)

    degp = _make_deg_kernel(NP, EP)(row, w, z1)

    y1t = _tc_pre(x.astype(f32), w1p, degp, NP)
    edge_k = _make_edge_kernel(NP, EP)
    S1 = edge_k(y1t.T, col, row, w, z8)
    y2t = _tc_mid(S1.T, y1t, degp, b1p, w2p, NP)

    S2 = edge_k(y2t.T, col, row, w, z8)
    out8t = _tc_post(S2.T, y2t, degp, b2p, NP)

    h = out8t.T[:N, :7]
    reg = jnp.array([0.0], dtype=h.dtype)
    return (h, reg)
